# baseline (speedup 1.0000x reference)
import jax
import jax.numpy as jnp
from jax import lax
from jax.experimental import pallas as pl
from jax.experimental.pallas import tpu as pltpu
from jax.experimental.pallas import tpu_sc as plsc

_N0 = 64
_N1 = 64
_VD = 64
_B = 16384
_NC = 2
_NS = 16
_NW = _NC * _NS
_BPW = _B // _NW
_CH = 16
_NCHUNK = _BPW // _CH


def _cr_w(s):
    s2 = s * s
    s3 = s2 * s
    w0 = 0.5 * (-s3 + 2.0 * s2 - s)
    w1 = 0.5 * (3.0 * s3 - 5.0 * s2 + 2.0)
    w2 = 0.5 * (-3.0 * s3 + 4.0 * s2 + s)
    w3 = 0.5 * (s3 - s2)
    return w0, w1, w2, w3


def _locate(xv, n):
    i = jnp.clip(xv.astype(jnp.int32), 1, n - 3)
    s = xv - i.astype(jnp.float32)
    return i, _cr_w(s)


def _tree(ps):
    while len(ps) > 1:
        nxt = [a + b for a, b in zip(ps[0::2], ps[1::2])]
        if len(ps) % 2:
            nxt.append(ps[-1])
        ps = nxt
    return ps[0]


def _body(x0_hbm, x1_hbm, tbl_hbm, out_hbm,
          x0_v, x1_v, idx_v, rows_v, out_v, sem):
    wid = lax.axis_index("s") * _NC + lax.axis_index("c")
    base = wid * _BPW
    pltpu.sync_copy(x0_hbm.at[pl.ds(base, _BPW)], x0_v)
    pltpu.sync_copy(x1_hbm.at[pl.ds(base, _BPW)], x1_v)

    def copy(par):
        return pltpu.make_async_copy(tbl_hbm.at[idx_v.at[par]],
                                     rows_v.at[par], sem)

    def prologue(cc, par):
        xv0 = x0_v[pl.ds(cc * _CH, _CH)]
        xv1 = x1_v[pl.ds(cc * _CH, _CH)]
        i0, _ = _locate(xv0, _N0)
        i1, _ = _locate(xv1, _N1)
        fi = (i0 - 1) * _N1 + (i1 - 1)
        for i in range(4):
            idx_v[par, pl.ds(i * _CH, _CH)] = fi + i * _N1
        copy(par).start()

    def accum(cc, par):
        xv0 = x0_v[pl.ds(cc * _CH, _CH)]
        xv1 = x1_v[pl.ds(cc * _CH, _CH)]
        _, w0 = _locate(xv0, _N0)
        _, w1 = _locate(xv1, _N1)
        wprod = [w0[r // 4] * w1[r % 4] for r in range(16)]
        for q in range(_CH):
            lo = []
            hi = []
            for r in range(16):
                ri, rj = r // 4, r % 4
                ws = jnp.broadcast_to(wprod[r][q], (16,))
                wb = plsc.pack(ws, ws, format=plsc.PackFormat.INTERLEAVED)
                row = rows_v.at[par, ri * _CH + q]
                lo.append(wb * row[pl.ds(rj * _VD, 32)])
                hi.append(wb * row[pl.ds(rj * _VD + 32, 32)])
            out_v[cc * _CH + q, pl.ds(0, 32)] = _tree(lo)
            out_v[cc * _CH + q, pl.ds(32, 32)] = _tree(hi)
        return 0

    prologue(0, 0)

    def pair(c2, _):
        c = c2 * 2
        copy(0).wait()
        prologue(c + 1, 1)
        accum(c, 0)
        copy(1).wait()
        prologue(jnp.minimum(c + 2, _NCHUNK - 1), 0)
        accum(c + 1, 1)
        return 0

    lax.fori_loop(0, _NCHUNK // 2, pair, 0)
    copy(0).wait()
    pltpu.sync_copy(out_v, out_hbm.at[pl.ds(base, _BPW)])


@jax.jit
def _sc_interp(x0, x1, tbl4):
    mesh = plsc.VectorSubcoreMesh(core_axis_name="c", subcore_axis_name="s")
    f = pl.kernel(
        _body,
        out_type=jax.ShapeDtypeStruct((_B, _VD), jnp.bfloat16),
        mesh=mesh,
        compiler_params=pltpu.CompilerParams(use_tc_tiling_on_sc=False,
                                             needs_layout_passes=False),
        scratch_types=[
            pltpu.VMEM((_BPW,), jnp.float32),
            pltpu.VMEM((_BPW,), jnp.float32),
            pltpu.VMEM((2, 64), jnp.int32),
            pltpu.VMEM((2, 64, 4 * _VD), jnp.bfloat16),
            pltpu.VMEM((_BPW, _VD), jnp.bfloat16),
            pltpu.SemaphoreType.DMA,
        ],
    )
    return f(x0, x1, tbl4)


def kernel(x, control_values, controls0, controls1):
    del controls0, controls1
    x0 = x[:, 0]
    x1 = x[:, 1]
    t = control_values.astype(jnp.bfloat16).reshape(_N0 * _N1, _VD)
    tbl4 = jnp.concatenate(
        [t, jnp.roll(t, -1, axis=0), jnp.roll(t, -2, axis=0),
         jnp.roll(t, -3, axis=0)], axis=1)
    return _sc_interp(x0, x1, tbl4).astype(jnp.float32)


# speedup vs baseline: 1.0567x; 1.0567x over previous
import jax
import jax.numpy as jnp
from jax import lax
from jax.experimental import pallas as pl
from jax.experimental.pallas import tpu as pltpu
from jax.experimental.pallas import tpu_sc as plsc

_N0 = 64
_N1 = 64
_VD = 64
_B = 16384
_NC = 2
_NS = 16
_NW = _NC * _NS
_BPW = _B // _NW
_CH = 16
_NCHUNK = _BPW // _CH


def _cr_w(s):
    s2 = s * s
    s3 = s2 * s
    w0 = 0.5 * (-s3 + 2.0 * s2 - s)
    w1 = 0.5 * (3.0 * s3 - 5.0 * s2 + 2.0)
    w2 = 0.5 * (-3.0 * s3 + 4.0 * s2 + s)
    w3 = 0.5 * (s3 - s2)
    return w0, w1, w2, w3


def _locate(xv, n):
    i = jnp.clip(xv.astype(jnp.int32), 1, n - 3)
    s = xv - i.astype(jnp.float32)
    return i, _cr_w(s)


def _tree(ps):
    while len(ps) > 1:
        nxt = [a + b for a, b in zip(ps[0::2], ps[1::2])]
        if len(ps) % 2:
            nxt.append(ps[-1])
        ps = nxt
    return ps[0]


def _body(xt_hbm, tbl_hbm, out_hbm,
          x0_v, x1_v, idx_v, rows_v, out_v, sem):
    wid = lax.axis_index("s") * _NC + lax.axis_index("c")
    base = wid * _BPW
    pltpu.sync_copy(xt_hbm.at[0, pl.ds(base, _BPW)], x0_v)
    pltpu.sync_copy(xt_hbm.at[1, pl.ds(base, _BPW)], x1_v)

    def copies(par):
        return (pltpu.make_async_copy(tbl_hbm.at[idx_v.at[par, 0]],
                                      rows_v.at[par, 0], sem),
                pltpu.make_async_copy(tbl_hbm.at[idx_v.at[par, 1]],
                                      rows_v.at[par, 1], sem))

    def prologue(cc, par):
        xv0 = x0_v[pl.ds(cc * _CH, _CH)]
        xv1 = x1_v[pl.ds(cc * _CH, _CH)]
        i0, _ = _locate(xv0, _N0)
        i1, _ = _locate(xv1, _N1)
        fi = (i0 - 1) * _N1 + (i1 - 1)
        for r in range(16):
            ri, rj = r // 4, r % 4
            idx_v[par, r // 8, pl.ds((r % 8) * _CH, _CH)] = fi + (ri * _N1 + rj)
        cp0, cp1 = copies(par)
        cp0.start()
        cp1.start()

    def wait(par):
        cp0, cp1 = copies(par)
        cp0.wait()
        cp1.wait()

    def accum(cc, par):
        xv0 = x0_v[pl.ds(cc * _CH, _CH)]
        xv1 = x1_v[pl.ds(cc * _CH, _CH)]
        _, w0 = _locate(xv0, _N0)
        _, w1 = _locate(xv1, _N1)
        wprod = [w0[r // 4] * w1[r % 4] for r in range(16)]
        for q in range(_CH):
            lo = []
            hi = []
            for r in range(16):
                ws = jnp.broadcast_to(wprod[r][q], (16,))
                wb = plsc.pack(ws, ws, format=plsc.PackFormat.INTERLEAVED)
                row = rows_v.at[par, r // 8, (r % 8) * _CH + q]
                lo.append(wb * row[pl.ds(0, 32)])
                hi.append(wb * row[pl.ds(32, 32)])
            out_v[cc * 8 + q // 2, pl.ds((q % 2) * 64, 32)] = _tree(lo)
            out_v[cc * 8 + q // 2, pl.ds((q % 2) * 64 + 32, 32)] = _tree(hi)
        return 0

    prologue(0, 0)

    def pair(c2, _):
        c = c2 * 2
        wait(0)
        prologue(c + 1, 1)
        accum(c, 0)
        wait(1)
        prologue(jnp.minimum(c + 2, _NCHUNK - 1), 0)
        accum(c + 1, 1)
        return 0

    lax.fori_loop(0, _NCHUNK // 2, pair, 0)
    wait(0)
    pltpu.sync_copy(out_v, out_hbm.at[pl.ds(wid * (_BPW // 2), _BPW // 2)])


@jax.jit
def _sc_interp(xt, tbl):
    mesh = plsc.VectorSubcoreMesh(core_axis_name="c", subcore_axis_name="s")
    f = pl.kernel(
        _body,
        out_type=jax.ShapeDtypeStruct((_B // 2, 2 * _VD), jnp.bfloat16),
        mesh=mesh,
        compiler_params=pltpu.CompilerParams(use_tc_tiling_on_sc=False, needs_layout_passes=False),
        scratch_types=[
            pltpu.VMEM((_BPW,), jnp.float32),
            pltpu.VMEM((_BPW,), jnp.float32),
            pltpu.VMEM((2, 2, 128), jnp.int32),
            pltpu.VMEM((2, 2, 128, _VD), jnp.bfloat16),
            pltpu.VMEM((_BPW // 2, 2 * _VD), jnp.bfloat16),
            pltpu.SemaphoreType.DMA,
        ],
    )
    return f(xt, tbl)


def kernel(x, control_values, controls0, controls1):
    del controls0, controls1
    tbl = control_values.reshape(_N0 * _N1, _VD).astype(jnp.bfloat16)
    out2 = _sc_interp(x.T, tbl)
    return out2.reshape(_B, _VD).astype(jnp.float32)


# speedup vs baseline: 1.0814x; 1.0234x over previous
import jax
import jax.numpy as jnp
from jax import lax
from jax.experimental import pallas as pl
from jax.experimental.pallas import tpu as pltpu
from jax.experimental.pallas import tpu_sc as plsc

_N0 = 64
_N1 = 64
_VD = 64
_B = 16384
_NC = 2
_NS = 16
_NW = _NC * _NS
_BPW = _B // _NW
_CH = 16
_NCHUNK = _BPW // _CH


def _cr_w(s):
    s2 = s * s
    s3 = s2 * s
    w0 = 0.5 * (-s3 + 2.0 * s2 - s)
    w1 = 0.5 * (3.0 * s3 - 5.0 * s2 + 2.0)
    w2 = 0.5 * (-3.0 * s3 + 4.0 * s2 + s)
    w3 = 0.5 * (s3 - s2)
    return w0, w1, w2, w3


def _locate(xv, n):
    i = jnp.clip(xv.astype(jnp.int32), 1, n - 3)
    s = xv - i.astype(jnp.float32)
    return i, _cr_w(s)


def _tree(ps):
    while len(ps) > 1:
        nxt = [a + b for a, b in zip(ps[0::2], ps[1::2])]
        if len(ps) % 2:
            nxt.append(ps[-1])
        ps = nxt
    return ps[0]


def _body(x0_hbm, x1_hbm, tbl_hbm, out_hbm,
          x0_v, x1_v, idx_v, rows_v, out_v, sem):
    wid = lax.axis_index("s") * _NC + lax.axis_index("c")
    base = wid * _BPW
    pltpu.sync_copy(x0_hbm.at[pl.ds(base, _BPW)], x0_v)
    pltpu.sync_copy(x1_hbm.at[pl.ds(base, _BPW)], x1_v)

    def copies(par):
        return (pltpu.make_async_copy(tbl_hbm.at[idx_v.at[par, 0]],
                                      rows_v.at[par, 0], sem),
                pltpu.make_async_copy(tbl_hbm.at[idx_v.at[par, 1]],
                                      rows_v.at[par, 1], sem))

    def prologue(cc, par):
        xv0 = x0_v[pl.ds(cc * _CH, _CH)]
        xv1 = x1_v[pl.ds(cc * _CH, _CH)]
        i0, _ = _locate(xv0, _N0)
        i1, _ = _locate(xv1, _N1)
        fi = (i0 - 1) * _N1 + (i1 - 1)
        for r in range(16):
            ri, rj = r // 4, r % 4
            idx_v[par, r // 8, pl.ds((r % 8) * _CH, _CH)] = fi + (ri * _N1 + rj)
        cp0, cp1 = copies(par)
        cp0.start()
        cp1.start()

    def wait(par):
        cp0, cp1 = copies(par)
        cp0.wait()
        cp1.wait()

    def accum(cc, par):
        xv0 = x0_v[pl.ds(cc * _CH, _CH)]
        xv1 = x1_v[pl.ds(cc * _CH, _CH)]
        _, w0 = _locate(xv0, _N0)
        _, w1 = _locate(xv1, _N1)
        wprod = [w0[r // 4] * w1[r % 4] for r in range(16)]
        for q in range(_CH):
            lo = []
            hi = []
            for r in range(16):
                ws = jnp.broadcast_to(wprod[r][q], (16,))
                wb = plsc.pack(ws, ws, format=plsc.PackFormat.INTERLEAVED)
                row = rows_v.at[par, r // 8, (r % 8) * _CH + q]
                lo.append(wb * row[pl.ds(0, 32)])
                hi.append(wb * row[pl.ds(32, 32)])
            out_v[cc * _CH + q, pl.ds(0, 32)] = _tree(lo)
            out_v[cc * _CH + q, pl.ds(32, 32)] = _tree(hi)
        return 0

    prologue(0, 0)

    def pair(c2, _):
        c = c2 * 2
        wait(0)
        prologue(c + 1, 1)
        accum(c, 0)
        wait(1)
        prologue(jnp.minimum(c + 2, _NCHUNK - 1), 0)
        accum(c + 1, 1)
        return 0

    lax.fori_loop(0, _NCHUNK // 2, pair, 0)
    wait(0)
    pltpu.sync_copy(out_v, out_hbm.at[pl.ds(base, _BPW)])


@jax.jit
def _sc_interp(x0, x1, tbl):
    mesh = plsc.VectorSubcoreMesh(core_axis_name="c", subcore_axis_name="s")
    f = pl.kernel(
        _body,
        out_type=jax.ShapeDtypeStruct((_B, _VD), jnp.bfloat16),
        mesh=mesh,
        compiler_params=pltpu.CompilerParams(use_tc_tiling_on_sc=False, needs_layout_passes=False),
        scratch_types=[
            pltpu.VMEM((_BPW,), jnp.float32),
            pltpu.VMEM((_BPW,), jnp.float32),
            pltpu.VMEM((2, 2, 128), jnp.int32),
            pltpu.VMEM((2, 2, 128, _VD), jnp.bfloat16),
            pltpu.VMEM((_BPW, _VD), jnp.bfloat16),
            pltpu.SemaphoreType.DMA,
        ],
    )
    return f(x0, x1, tbl)


def kernel(x, control_values, controls0, controls1):
    del controls0, controls1
    x0 = x[:, 0]
    x1 = x[:, 1]
    tbl = control_values.reshape(_N0 * _N1, _VD).astype(jnp.bfloat16)
    return _sc_interp(x0, x1, tbl).astype(jnp.float32)
